# baseline (speedup 1.0000x reference)
import jax
import jax.numpy as jnp
from jax.experimental import pallas as pl
from jax.experimental.pallas import tpu as pltpu

_B, _T = 2, 2048
_D = 1024
_E = 8
_TN = 512


def _moe_kernel(x_ref, wg_ref, we_hbm, out_ref, w_vmem, sems):
    i = pl.program_id(0)

    def _w_copy(e):
        return pltpu.make_async_copy(
            we_hbm.at[:, pl.ds(e * _D, _D)],
            w_vmem.at[:, pl.ds(e * _D, _D)],
            sems.at[e])

    @pl.when(i == 0)
    def _():
        for e in range(_E):
            _w_copy(e).start()

    x = x_ref[...]

    g = jnp.dot(x, wg_ref[...], preferred_element_type=jnp.float32)
    sm = jax.nn.softmax(g, axis=-1)

    e_ids = jax.lax.broadcasted_iota(jnp.int32, g.shape, 1)
    a1 = jnp.argmax(g, axis=-1, keepdims=True)
    m1 = e_ids == a1
    g2 = jnp.where(m1, -jnp.inf, g)
    a2 = jnp.argmax(g2, axis=-1, keepdims=True)
    m2 = e_ids == a2
    sc = jnp.where(m1 | m2, sm, 0.0)

    acc = jnp.zeros(out_ref.shape, jnp.float32)
    for e in range(_E):
        @pl.when(i == 0)
        def _(e=e):
            _w_copy(e).wait()

        s_e = sc[:, e][:, None]
        acc += s_e * jnp.dot(x, w_vmem[:, e * _D:(e + 1) * _D],
                             preferred_element_type=jnp.float32)
    out_ref[...] = acc


@jax.jit
def kernel(x, W_gate, W_exp):
    n = _B * _T
    xf = x.reshape(n, _D)
    out = pl.pallas_call(
        _moe_kernel,
        grid=(n // _TN,),
        in_specs=[
            pl.BlockSpec((_TN, _D), lambda i: (i, 0)),
            pl.BlockSpec((_D, _E), lambda i: (0, 0)),
            pl.BlockSpec(memory_space=pl.ANY),
        ],
        out_specs=pl.BlockSpec((_TN, _D), lambda i: (i, 0)),
        out_shape=jax.ShapeDtypeStruct((n, _D), jnp.float32),
        scratch_shapes=[
            pltpu.VMEM((_D, _E * _D), jnp.float32),
            pltpu.SemaphoreType.DMA((_E,)),
        ],
    )(xf, W_gate, W_exp)
    return out.reshape(_B, _T, _D)


# speedup vs baseline: 1.0984x; 1.0984x over previous
import jax
import jax.numpy as jnp
from jax.experimental import pallas as pl
from jax.experimental.pallas import tpu as pltpu

_B, _T = 2, 2048
_D = 1024
_E = 8
_TN = 1024


def _moe_kernel(x_ref, wg_ref, we_hbm, out_ref, w_vmem, sems):
    i = pl.program_id(0)

    def _w_copy(e):
        return pltpu.make_async_copy(
            we_hbm.at[:, pl.ds(e * _D, _D)],
            w_vmem.at[:, pl.ds(e * _D, _D)],
            sems.at[e])

    @pl.when(i == 0)
    def _():
        for e in range(_E):
            _w_copy(e).start()

    x = x_ref[...]

    g = jnp.dot(x, wg_ref[...], preferred_element_type=jnp.float32)
    sm = jax.nn.softmax(g, axis=-1)

    e_ids = jax.lax.broadcasted_iota(jnp.int32, g.shape, 1)
    a1 = jnp.argmax(g, axis=-1, keepdims=True)
    m1 = e_ids == a1
    g2 = jnp.where(m1, -jnp.inf, g)
    a2 = jnp.argmax(g2, axis=-1, keepdims=True)
    m2 = e_ids == a2
    sc = jnp.where(m1 | m2, sm, 0.0)

    acc = jnp.zeros(out_ref.shape, jnp.float32)
    for e in range(_E):
        @pl.when(i == 0)
        def _(e=e):
            _w_copy(e).wait()

        s_e = sc[:, e][:, None]
        acc += s_e * jnp.dot(x, w_vmem[:, e * _D:(e + 1) * _D],
                             preferred_element_type=jnp.float32)
    out_ref[...] = acc


@jax.jit
def kernel(x, W_gate, W_exp):
    n = _B * _T
    xf = x.reshape(n, _D)
    out = pl.pallas_call(
        _moe_kernel,
        grid=(n // _TN,),
        in_specs=[
            pl.BlockSpec((_TN, _D), lambda i: (i, 0)),
            pl.BlockSpec((_D, _E), lambda i: (0, 0)),
            pl.BlockSpec(memory_space=pl.ANY),
        ],
        out_specs=pl.BlockSpec((_TN, _D), lambda i: (i, 0)),
        out_shape=jax.ShapeDtypeStruct((n, _D), jnp.float32),
        scratch_shapes=[
            pltpu.VMEM((_D, _E * _D), jnp.float32),
            pltpu.SemaphoreType.DMA((_E,)),
        ],
    )(xf, W_gate, W_exp)
    return out.reshape(_B, _T, _D)
